# baseline (speedup 1.0000x reference)
import functools

import jax
import jax.numpy as jnp
from jax import lax
from jax.experimental import pallas as pl
from jax.experimental.pallas import tpu as pltpu
from jax.experimental.pallas import tpu_sc as plsc

N = 100000
NUM_SEG = 1024
CHUNK = 6272
NCHUNK = 16
GRID = 2
XBLK = 8 * CHUNK
EROW = GRID * CHUNK
LAST_CNT = N - (NCHUNK - 1) * CHUNK
G_FULL = CHUNK // 16
G_LAST = LAST_CNT // 16
SEG_PER_W = NUM_SEG // NCHUNK
FBLK = 784


def _mlp_body(x_ref, wa_ref, b1_ref, w2t_ref, b2_ref, o_ref):
    h = jnp.dot(x_ref[pl.ds(0, CHUNK), :], wa_ref[pl.ds(0, 64), :],
                preferred_element_type=jnp.float32)
    for c in range(1, 8):
        h = h + jnp.dot(x_ref[pl.ds(c * CHUNK, CHUNK), :],
                        wa_ref[pl.ds(c * 64, 64), :],
                        preferred_element_type=jnp.float32)
    h = h + b1_ref[...]
    sil = h * (1.0 / (1.0 + jnp.exp(-h)))
    e8t = lax.dot_general(w2t_ref[...], sil, (((1,), (1,)), ((), ())),
                          preferred_element_type=jnp.float32)
    o_ref[...] = e8t + b2_ref[...]


def _mlp(x, wa, b1t, w2t, b2t):
    return pl.pallas_call(
        _mlp_body,
        grid=(GRID,),
        in_specs=[
            pl.BlockSpec((XBLK, 64), lambda i: (i, 0)),
            pl.BlockSpec((512, 128), lambda i: (0, 0)),
            pl.BlockSpec((1, 128), lambda i: (0, 0)),
            pl.BlockSpec((8, 128), lambda i: (0, 0)),
            pl.BlockSpec((1, 1), lambda i: (0, 0)),
        ],
        out_specs=pl.BlockSpec((8, CHUNK), lambda i: (0, i)),
        out_shape=jax.ShapeDtypeStruct((8, EROW), jnp.float32),
    )(x, wa, b1t, w2t, b2t)


def _seg_body(e_hbm, data_hbm, zin_hbm, out_hbm, f_hbm, ev, idv, bins,
              partial, red, seg_out, shared, sem):
    sid = lax.axis_index("s")
    ci = sid % 8
    gi = sid // 8
    ebase = gi * CHUNK
    nbase = sid * CHUNK
    is_last = sid == NCHUNK - 1
    ngroups = jnp.where(is_last, G_LAST, G_FULL)

    cp_e = pltpu.make_async_copy(e_hbm.at[ci, pl.ds(ebase, CHUNK)], ev, sem)
    cp_i = pltpu.make_async_copy(data_hbm.at[pl.ds(nbase, CHUNK)], idv, sem)
    cp_e_l = pltpu.make_async_copy(e_hbm.at[ci, pl.ds(ebase, LAST_CNT)],
                                   ev.at[pl.ds(0, LAST_CNT)], sem)
    cp_i_l = pltpu.make_async_copy(data_hbm.at[pl.ds(nbase, LAST_CNT)],
                                   idv.at[pl.ds(0, LAST_CNT)], sem)

    @pl.when(jnp.logical_not(is_last))
    def _():
        cp_e.start()
        cp_i.start()

    @pl.when(is_last)
    def _():
        cp_e_l.start()
        cp_i_l.start()

    lanes = lax.iota(jnp.int32, 16)
    zero16 = jnp.zeros((16,), jnp.float32)

    @pl.when(jnp.logical_not(is_last))
    def _():
        pltpu.sync_copy(zin_hbm, f_hbm.at[pl.ds(nbase, CHUNK)])

    @pl.when(is_last)
    def _():
        pltpu.sync_copy(zin_hbm.at[pl.ds(0, LAST_CNT)],
                        f_hbm.at[pl.ds(nbase, LAST_CNT)])

    def _z(j, _):
        for r in range(16):
            bins[r, pl.ds(j * 16, 16)] = zero16
        return 0
    lax.fori_loop(0, NUM_SEG // 16, _z, 0)

    @pl.when(jnp.logical_not(is_last))
    def _():
        cp_e.wait()
        cp_i.wait()

    @pl.when(is_last)
    def _():
        cp_e_l.wait()
        cp_i_l.wait()

    def _group(g, _):
        row0 = g * 16
        e = ev[pl.ds(row0, 16)]
        ids = idv[pl.ds(row0, 16)]
        plsc.addupdate_scatter(bins, [lanes, ids], e)
        return 0

    lax.fori_loop(0, ngroups, _group, 0)

    def _red(gj, _):
        c0 = gj * 16
        acc = bins[0, pl.ds(c0, 16)]
        for r in range(1, 16):
            acc = acc + bins[r, pl.ds(c0, 16)]
        partial[pl.ds(c0, 16)] = acc
        return 0
    lax.fori_loop(0, NUM_SEG // 16, _red, 0)

    pltpu.sync_copy(partial, shared.at[sid])
    plsc.subcore_barrier()
    c0 = sid * SEG_PER_W
    pltpu.sync_copy(shared.at[:, pl.ds(c0, SEG_PER_W)], red)
    for j in range(SEG_PER_W // 16):
        acc = red[0, pl.ds(j * 16, 16)]
        for r in range(1, 16):
            acc = acc + red[r, pl.ds(j * 16, 16)]
        seg_out[pl.ds(j * 16, 16)] = acc
    pltpu.sync_copy(seg_out, out_hbm.at[pl.ds(c0, SEG_PER_W)])


@functools.partial(
    pl.kernel,
    mesh=plsc.VectorSubcoreMesh(core_axis_name="c", subcore_axis_name="s",
                                num_cores=1),
    out_type=(jax.ShapeDtypeStruct((NUM_SEG,), jnp.float32),
              jax.ShapeDtypeStruct((N, 3), jnp.float32)),
    scratch_types=[
        pltpu.VMEM((CHUNK,), jnp.float32),
        pltpu.VMEM((CHUNK,), jnp.int32),
        pltpu.VMEM((16, NUM_SEG), jnp.float32),
        pltpu.VMEM((NUM_SEG,), jnp.float32),
        pltpu.VMEM((NCHUNK, SEG_PER_W), jnp.float32),
        pltpu.VMEM((SEG_PER_W,), jnp.float32),
        pltpu.VMEM_SHARED((NCHUNK, NUM_SEG), jnp.float32),
        pltpu.SemaphoreType.DMA,
    ],
    compiler_params=pltpu.CompilerParams(use_tc_tiling_on_sc=False,
                                         needs_layout_passes=False),
)
def _sc_segsum(e_hbm, data_hbm, zin_hbm, out_hbm, f_hbm, ev, idv, bins,
               partial, red, seg_out, shared, sem):
    _seg_body(e_hbm, data_hbm, zin_hbm, out_hbm, f_hbm, ev, idv, bins,
              partial, red, seg_out, shared, sem)


def kernel(x, data, pos, W1, b1, W2, b2):
    data_i = data.astype(jnp.int32)
    w1blk = jnp.zeros((64, 16), jnp.float32).at[48:64, :].set(
        W1.T.astype(jnp.float32))
    eye = jnp.eye(8, dtype=jnp.float32)
    wa = jnp.kron(eye, w1blk)
    b1t = jnp.tile(b1.astype(jnp.float32), 8).reshape(1, 128)
    w2t = jnp.kron(eye, W2.astype(jnp.float32).reshape(1, 16))
    b2t = b2.astype(jnp.float32).reshape(1, 1)

    e8t = _mlp(x, wa, b1t, w2t, b2t)
    zin = jnp.zeros((CHUNK, 3), jnp.float32)
    E, F = _sc_segsum(e8t, data_i, zin)
    return (E.reshape(NUM_SEG, 1), F)


# speedup vs baseline: 2.7899x; 2.7899x over previous
import functools

import jax
import jax.numpy as jnp
from jax import lax
from jax.experimental import pallas as pl
from jax.experimental.pallas import tpu as pltpu
from jax.experimental.pallas import tpu_sc as plsc

N = 100000
NUM_SEG = 1024
CHUNK = 6272
NCHUNK = 16
GRID = 2
XBLK = 8 * CHUNK
EROW = GRID * CHUNK
LAST_CNT = N - (NCHUNK - 1) * CHUNK
G_FULL = CHUNK // 16
G_LAST = LAST_CNT // 16
SEG_PER_W = NUM_SEG // NCHUNK
FBLK = 784


def _mlp_body(x_ref, wa_ref, b1_ref, w2t_ref, b2_ref, o_ref):
    h = jnp.dot(x_ref[pl.ds(0, CHUNK), :], wa_ref[pl.ds(0, 64), :],
                preferred_element_type=jnp.float32)
    for c in range(1, 8):
        h = h + jnp.dot(x_ref[pl.ds(c * CHUNK, CHUNK), :],
                        wa_ref[pl.ds(c * 64, 64), :],
                        preferred_element_type=jnp.float32)
    h = h + b1_ref[...]
    sil = h * (1.0 / (1.0 + jnp.exp(-h)))
    e8t = lax.dot_general(w2t_ref[...], sil, (((1,), (1,)), ((), ())),
                          preferred_element_type=jnp.float32)
    o_ref[...] = e8t + b2_ref[...]


def _mlp(x, wa, b1t, w2t, b2t):
    return pl.pallas_call(
        _mlp_body,
        grid=(GRID,),
        in_specs=[
            pl.BlockSpec((XBLK, 64), lambda i: (i, 0)),
            pl.BlockSpec((512, 128), lambda i: (0, 0)),
            pl.BlockSpec((1, 128), lambda i: (0, 0)),
            pl.BlockSpec((8, 128), lambda i: (0, 0)),
            pl.BlockSpec((1, 1), lambda i: (0, 0)),
        ],
        out_specs=pl.BlockSpec((8, CHUNK), lambda i: (0, i)),
        out_shape=jax.ShapeDtypeStruct((8, EROW), jnp.float32),
    )(x, wa, b1t, w2t, b2t)


def _seg_body(e_hbm, data_hbm, out_hbm, ev, idv, bins,
              partial, red, seg_out, shared, sem):
    sid = lax.axis_index("s")
    ci = sid % 8
    gi = sid // 8
    ebase = gi * CHUNK
    nbase = sid * CHUNK
    is_last = sid == NCHUNK - 1
    ngroups = jnp.where(is_last, G_LAST, G_FULL)

    cp_e = pltpu.make_async_copy(e_hbm.at[ci, pl.ds(ebase, CHUNK)], ev, sem)
    cp_i = pltpu.make_async_copy(data_hbm.at[pl.ds(nbase, CHUNK)], idv, sem)
    cp_e_l = pltpu.make_async_copy(e_hbm.at[ci, pl.ds(ebase, LAST_CNT)],
                                   ev.at[pl.ds(0, LAST_CNT)], sem)
    cp_i_l = pltpu.make_async_copy(data_hbm.at[pl.ds(nbase, LAST_CNT)],
                                   idv.at[pl.ds(0, LAST_CNT)], sem)

    @pl.when(jnp.logical_not(is_last))
    def _():
        cp_e.start()
        cp_i.start()

    @pl.when(is_last)
    def _():
        cp_e_l.start()
        cp_i_l.start()

    lanes = lax.iota(jnp.int32, 16)
    zero16 = jnp.zeros((16,), jnp.float32)

    def _z(j, _):
        for r in range(16):
            bins[r, pl.ds(j * 16, 16)] = zero16
        return 0
    lax.fori_loop(0, NUM_SEG // 16, _z, 0)

    @pl.when(jnp.logical_not(is_last))
    def _():
        cp_e.wait()
        cp_i.wait()

    @pl.when(is_last)
    def _():
        cp_e_l.wait()
        cp_i_l.wait()

    def _group(g, _):
        row0 = g * 16
        e = ev[pl.ds(row0, 16)]
        ids = idv[pl.ds(row0, 16)]
        plsc.addupdate_scatter(bins, [lanes, ids], e)
        return 0

    lax.fori_loop(0, ngroups, _group, 0)

    def _red(gj, _):
        c0 = gj * 16
        acc = bins[0, pl.ds(c0, 16)]
        for r in range(1, 16):
            acc = acc + bins[r, pl.ds(c0, 16)]
        partial[pl.ds(c0, 16)] = acc
        return 0
    lax.fori_loop(0, NUM_SEG // 16, _red, 0)

    pltpu.sync_copy(partial, shared.at[sid])
    plsc.subcore_barrier()
    c0 = sid * SEG_PER_W
    pltpu.sync_copy(shared.at[:, pl.ds(c0, SEG_PER_W)], red)
    for j in range(SEG_PER_W // 16):
        acc = red[0, pl.ds(j * 16, 16)]
        for r in range(1, 16):
            acc = acc + red[r, pl.ds(j * 16, 16)]
        seg_out[pl.ds(j * 16, 16)] = acc
    pltpu.sync_copy(seg_out, out_hbm.at[pl.ds(c0, SEG_PER_W)])


@functools.partial(
    pl.kernel,
    mesh=plsc.VectorSubcoreMesh(core_axis_name="c", subcore_axis_name="s",
                                num_cores=1),
    out_type=jax.ShapeDtypeStruct((NUM_SEG,), jnp.float32),
    scratch_types=[
        pltpu.VMEM((CHUNK,), jnp.float32),
        pltpu.VMEM((CHUNK,), jnp.int32),
        pltpu.VMEM((16, NUM_SEG), jnp.float32),
        pltpu.VMEM((NUM_SEG,), jnp.float32),
        pltpu.VMEM((NCHUNK, SEG_PER_W), jnp.float32),
        pltpu.VMEM((SEG_PER_W,), jnp.float32),
        pltpu.VMEM_SHARED((NCHUNK, NUM_SEG), jnp.float32),
        pltpu.SemaphoreType.DMA,
    ],
    compiler_params=pltpu.CompilerParams(use_tc_tiling_on_sc=False,
                                         needs_layout_passes=False),
)
def _sc_segsum(e_hbm, data_hbm, out_hbm, ev, idv, bins,
               partial, red, seg_out, shared, sem):
    _seg_body(e_hbm, data_hbm, out_hbm, ev, idv, bins,
              partial, red, seg_out, shared, sem)


def kernel(x, data, pos, W1, b1, W2, b2):
    data_i = data.astype(jnp.int32)
    w1blk = jnp.zeros((64, 16), jnp.float32).at[48:64, :].set(
        W1.T.astype(jnp.float32))
    eye = jnp.eye(8, dtype=jnp.float32)
    wa = jnp.kron(eye, w1blk)
    b1t = jnp.tile(b1.astype(jnp.float32), 8).reshape(1, 128)
    w2t = jnp.kron(eye, W2.astype(jnp.float32).reshape(1, 16))
    b2t = b2.astype(jnp.float32).reshape(1, 1)

    e8t = _mlp(x, wa, b1t, w2t, b2t)
    E = _sc_segsum(e8t, data_i)
    F = jnp.zeros((N, 3), jnp.float32)
    return (E.reshape(NUM_SEG, 1), F)
